# baseline (speedup 1.0000x reference)
import functools

import jax
import jax.numpy as jnp
from jax import lax
from jax.experimental import pallas as pl
from jax.experimental.pallas import tpu as pltpu
from jax.experimental.pallas import tpu_sc as plsc

VOCAB = 1000000
DIM = 32
N_NODES = 100000

_BR = 2048
_BC = 4 * _BR
_NBLK = -(-VOCAB // _BC)
_NFULL = VOCAB // _BC
_PROWS = _NBLK * _BR

_info = plsc.get_sparse_core_info()
_NC, _NS = _info.num_cores, _info.num_subcores
_NW = _NC * _NS

_CHUNK = 640
_NCHUNK = 5
_B_PER_W = _CHUNK * _NCHUNK
_B_PAD = _B_PER_W * _NW


def _tc_repack(b0_ref, b1_ref, b2_ref, b3_ref, t0_ref, t1_ref, t2_ref,
               t3_ref, out_ref):
    i = pl.program_id(0)

    @pl.when(i < _NFULL)
    def _main():
        out_ref[...] = jnp.concatenate(
            [b0_ref[...], b1_ref[...], b2_ref[...], b3_ref[...]], axis=0).T

    @pl.when(i == _NFULL)
    def _tail():
        out_ref[...] = jnp.concatenate(
            [t0_ref[...], t1_ref[...], t2_ref[...], t3_ref[...]], axis=0).T


_repack = pl.pallas_call(
    _tc_repack,
    grid=(_NBLK,),
    in_specs=[
        pl.BlockSpec(
            (DIM, _BR),
            functools.partial(
                lambda s, i: (0, jnp.where(i < _NFULL, 4 * i + s, 0)), s))
        for s in range(4)
    ] + [
        pl.BlockSpec(
            (DIM, _BR),
            functools.partial(
                lambda s, i: (0, jnp.where(i < _NFULL, 0, s)), s))
        for s in range(4)
    ],
    out_specs=pl.BlockSpec((_BR, 128), lambda i: (i, 0)),
    out_shape=jax.ShapeDtypeStruct((_PROWS, 128), jnp.float32),
)


def _make_sc_gather():
    mesh = plsc.VectorSubcoreMesh(core_axis_name="c", subcore_axis_name="s")

    @functools.partial(
        pl.kernel,
        mesh=mesh,
        out_type=jax.ShapeDtypeStruct((_B_PAD, DIM), jnp.float32),
        scratch_types=[
            pltpu.VMEM((_CHUNK,), jnp.int32),
            pltpu.VMEM((_CHUNK,), jnp.int32),
            pltpu.VMEM((_CHUNK, 128), jnp.float32),
            pltpu.VMEM((_CHUNK, DIM), jnp.float32),
            pltpu.SemaphoreType.DMA,
        ],
        compiler_params=pltpu.CompilerParams(
            use_tc_tiling_on_sc=False, needs_layout_passes=False),
    )
    def sc_gather(packed_hbm, idx_hbm, out_hbm, idx_v, row_v, wide_v, out_v,
                  sem):
        wid = lax.axis_index("s") * _NC + lax.axis_index("c")
        base = wid * _B_PER_W

        def do_chunk(chunk_base):
            pltpu.sync_copy(idx_hbm.at[pl.ds(chunk_base, _CHUNK)], idx_v)

            @pl.loop(0, _CHUNK // 16)
            def _rows(i):
                t = idx_v[pl.ds(i * 16, 16)]
                row_v[pl.ds(i * 16, 16)] = ((t >> 13) << 11) + (t & 2047)

            pltpu.async_copy(packed_hbm.at[row_v], wide_v, sem).wait()

            @pl.loop(0, _CHUNK // 16)
            def _extract(g):
                t = idx_v[pl.ds(g * 16, 16)]
                rowloc = lax.iota(jnp.int32, 16) + g * 16
                colbase = ((t >> 11) & 3) << 5
                for j in range(DIM):
                    vals = plsc.load_gather(wide_v, [rowloc, colbase + j])
                    plsc.store_scatter(
                        out_v, [rowloc, jnp.full((16,), j, jnp.int32)], vals)

            pltpu.sync_copy(out_v, out_hbm.at[pl.ds(chunk_base, _CHUNK)])

        for k in range(_NCHUNK):
            do_chunk(base + k * _CHUNK)

    return sc_gather


_sc_gather = _make_sc_gather()


def kernel(token_id, table):
    idx = jnp.pad(token_id, (0, _B_PAD - N_NODES))
    tt = table.T
    tail_tt = jnp.pad(tt[:, _NFULL * _BC:], ((0, 0), (0, (_NFULL + 1) * _BC - VOCAB)))
    packed = _repack(tt, tt, tt, tt, tail_tt, tail_tt, tail_tt, tail_tt)
    out = _sc_gather(packed, idx)
    return out[:N_NODES]


# speedup vs baseline: 1.6711x; 1.6711x over previous
import functools

import jax
import jax.numpy as jnp
from jax import lax
from jax.experimental import pallas as pl
from jax.experimental.pallas import tpu as pltpu
from jax.experimental.pallas import tpu_sc as plsc

VOCAB = 1000000
DIM = 32
N_NODES = 100000

_BR = 2048
_BC = 4 * _BR
_NBLK = -(-VOCAB // _BC)
_NFULL = VOCAB // _BC
_PROWS = _NBLK * _BR

_info = plsc.get_sparse_core_info()
_NC, _NS = _info.num_cores, _info.num_subcores
_NW = _NC * _NS

_CHUNK = 640
_NCHUNK = 5
_B_PER_W = _CHUNK * _NCHUNK
_B_PAD = _B_PER_W * _NW


def _tc_repack(b0_ref, b1_ref, b2_ref, b3_ref, t0_ref, t1_ref, t2_ref,
               t3_ref, out_ref):
    i = pl.program_id(0)

    @pl.when(i < _NFULL)
    def _main():
        out_ref[...] = jnp.concatenate(
            [b0_ref[...], b1_ref[...], b2_ref[...], b3_ref[...]], axis=0).T

    @pl.when(i == _NFULL)
    def _tail():
        out_ref[...] = jnp.concatenate(
            [t0_ref[...], t1_ref[...], t2_ref[...], t3_ref[...]], axis=0).T


_repack = pl.pallas_call(
    _tc_repack,
    grid=(_NBLK,),
    in_specs=[
        pl.BlockSpec(
            (DIM, _BR),
            functools.partial(
                lambda s, i: (0, jnp.where(i < _NFULL, 4 * i + s, 0)), s))
        for s in range(4)
    ] + [
        pl.BlockSpec(
            (DIM, _BR),
            functools.partial(
                lambda s, i: (0, jnp.where(i < _NFULL, 0, s)), s))
        for s in range(4)
    ],
    out_specs=pl.BlockSpec((_BR, 128), lambda i: (i, 0)),
    out_shape=jax.ShapeDtypeStruct((_PROWS, 128), jnp.float32),
)


def _make_sc_gather():
    mesh = plsc.VectorSubcoreMesh(core_axis_name="c", subcore_axis_name="s")

    @functools.partial(
        pl.kernel,
        mesh=mesh,
        out_type=jax.ShapeDtypeStruct((_B_PAD, DIM), jnp.float32),
        scratch_types=[
            pltpu.VMEM((_CHUNK,), jnp.int32),
            pltpu.VMEM((_CHUNK,), jnp.int32),
            pltpu.VMEM((_CHUNK, DIM), jnp.float32),
            pltpu.SemaphoreType.DMA,
        ],
        compiler_params=pltpu.CompilerParams(use_tc_tiling_on_sc=False),
    )
    def sc_gather(packed_hbm, idx_hbm, out_hbm, idx_v, row_v, rows_v, sem):
        wid = lax.axis_index("s") * _NC + lax.axis_index("c")
        base = wid * _B_PER_W

        def do_chunk(chunk_base):
            pltpu.sync_copy(idx_hbm.at[pl.ds(chunk_base, _CHUNK)], idx_v)

            @pl.loop(0, _CHUNK // 16)
            def _rows(i):
                t = idx_v[pl.ds(i * 16, 16)]
                row_v[pl.ds(i * 16, 16)] = (((t >> 13) << 13) + ((t & 2047) << 2)
                                            + ((t >> 11) & 3))

            pltpu.async_copy(packed_hbm.at[row_v], rows_v, sem).wait()
            pltpu.sync_copy(rows_v, out_hbm.at[pl.ds(chunk_base, _CHUNK)])

        for k in range(_NCHUNK):
            do_chunk(base + k * _CHUNK)

    return sc_gather


_sc_gather = _make_sc_gather()


def kernel(token_id, table):
    idx = jnp.pad(token_id, (0, _B_PAD - N_NODES))
    tt = table.T
    tail_tt = jnp.pad(tt[:, _NFULL * _BC:], ((0, 0), (0, (_NFULL + 1) * _BC - VOCAB)))
    packed = _repack(tt, tt, tt, tt, tail_tt, tail_tt, tail_tt, tail_tt)
    packed32 = packed.reshape(_PROWS * 4, DIM)
    out = _sc_gather(packed32, idx)
    return out[:N_NODES]


# speedup vs baseline: 2.6441x; 1.5823x over previous
import functools

import jax
import jax.numpy as jnp
from jax import lax
from jax.experimental import pallas as pl
from jax.experimental.pallas import tpu as pltpu
from jax.experimental.pallas import tpu_sc as plsc

VOCAB = 1000000
DIM = 32
N_NODES = 100000

_BR = 4096
_BC = 4 * _BR
_NBLK = -(-VOCAB // _BC)
_NFULL = VOCAB // _BC
_PROWS = _NBLK * _BR

_info = plsc.get_sparse_core_info()
_NC, _NS = _info.num_cores, _info.num_subcores
_NW = _NC * _NS

_B_PER_W = ((N_NODES + _NW - 1) // _NW + 7) // 8 * 8
_B_LAST = N_NODES - (_NW - 1) * _B_PER_W


def _tc_repack(b0_ref, b1_ref, b2_ref, b3_ref, t0_ref, t1_ref, t2_ref,
               t3_ref, out_ref):
    i = pl.program_id(0)

    @pl.when(i < _NFULL)
    def _main():
        out_ref[...] = jnp.concatenate(
            [b0_ref[...], b1_ref[...], b2_ref[...], b3_ref[...]], axis=0).T

    @pl.when(i == _NFULL)
    def _tail():
        out_ref[...] = jnp.concatenate(
            [t0_ref[...], t1_ref[...], t2_ref[...], t3_ref[...]], axis=0).T


_repack = pl.pallas_call(
    _tc_repack,
    grid=(_NBLK,),
    in_specs=[
        pl.BlockSpec(
            (DIM, _BR),
            functools.partial(
                lambda s, i: (0, jnp.where(i < _NFULL, 4 * i + s, 0)), s))
        for s in range(4)
    ] + [
        pl.BlockSpec(
            (DIM, _BR),
            functools.partial(
                lambda s, i: (0, jnp.where(i < _NFULL, 0, s)), s))
        for s in range(4)
    ],
    out_specs=pl.BlockSpec((_BR, 128), lambda i: (i, 0)),
    out_shape=jax.ShapeDtypeStruct((_PROWS, 128), jnp.float32),
)


def _make_sc_gather():
    mesh = plsc.VectorSubcoreMesh(core_axis_name="c", subcore_axis_name="s")

    @functools.partial(
        pl.kernel,
        mesh=mesh,
        out_type=jax.ShapeDtypeStruct((N_NODES, DIM), jnp.float32),
        scratch_types=[
            pltpu.VMEM((_B_PER_W,), jnp.int32),
            pltpu.VMEM((_B_PER_W,), jnp.int32),
            pltpu.VMEM((_B_PER_W, DIM), jnp.float32),
            pltpu.SemaphoreType.DMA,
        ],
        compiler_params=pltpu.CompilerParams(use_tc_tiling_on_sc=False),
    )
    def sc_gather(packed_hbm, idx_hbm, out_hbm, idx_v, row_v, rows_v, sem):
        wid = lax.axis_index("s") * _NC + lax.axis_index("c")
        base = wid * _B_PER_W

        def do_slice(n):
            pltpu.sync_copy(idx_hbm.at[pl.ds(base, n)], idx_v.at[pl.ds(0, n)])

            @pl.loop(0, -(-n // 16))
            def _rows(i):
                off = jnp.minimum(i * 16, n - 16)
                t = idx_v[pl.ds(off, 16)]
                row_v[pl.ds(off, 16)] = (((t >> 14) << 14) + ((t & 4095) << 2)
                                         + ((t >> 12) & 3))

            pltpu.async_copy(packed_hbm.at[row_v.at[pl.ds(0, n)]],
                             rows_v.at[pl.ds(0, n)], sem).wait()
            pltpu.sync_copy(rows_v.at[pl.ds(0, n)],
                            out_hbm.at[pl.ds(base, n)])

        @pl.when(wid < _NW - 1)
        def _full():
            do_slice(_B_PER_W)

        @pl.when(wid == _NW - 1)
        def _last():
            do_slice(_B_LAST)

    return sc_gather


_sc_gather = _make_sc_gather()


def kernel(token_id, table):
    tt = table.T
    tail_tt = jnp.pad(tt[:, _NFULL * _BC:], ((0, 0), (0, (_NFULL + 1) * _BC - VOCAB)))
    packed = _repack(tt, tt, tt, tt, tail_tt, tail_tt, tail_tt, tail_tt)
    packed32 = packed.reshape(_PROWS * 4, DIM)
    return _sc_gather(packed32, token_id)


# speedup vs baseline: 3.3615x; 1.2713x over previous
import functools

import jax
import jax.numpy as jnp
from jax import lax
from jax.experimental import pallas as pl
from jax.experimental.pallas import tpu as pltpu
from jax.experimental.pallas import tpu_sc as plsc

VOCAB = 1000000
DIM = 32
N_NODES = 100000

_BR = 4096
_BC = 4 * _BR
_NBLK = -(-VOCAB // _BC)
_NFULL = VOCAB // _BC
_PROWS = _NBLK * _BR

_info = plsc.get_sparse_core_info()
_NC, _NS = _info.num_cores, _info.num_subcores
_NW = _NC * _NS

_B_PER_W = ((N_NODES + _NW - 1) // _NW + 31) // 32 * 32
_B_LAST = N_NODES - (_NW - 1) * _B_PER_W
_OROWS = (_NW // 4) * _B_PER_W
_FBLK = 4 * _B_PER_W


def _tc_repack(b0_ref, b1_ref, b2_ref, b3_ref, t0_ref, t1_ref, t2_ref,
               t3_ref, out_ref):
    i = pl.program_id(0)

    @pl.when(i < _NFULL)
    def _main():
        out_ref[...] = jnp.concatenate(
            [b0_ref[...], b1_ref[...], b2_ref[...], b3_ref[...]], axis=0).T

    @pl.when(i == _NFULL)
    def _tail():
        out_ref[...] = jnp.concatenate(
            [t0_ref[...], t1_ref[...], t2_ref[...], t3_ref[...]], axis=0).T


_repack = pl.pallas_call(
    _tc_repack,
    grid=(_NBLK,),
    in_specs=[
        pl.BlockSpec(
            (DIM, _BR),
            functools.partial(
                lambda s, i: (0, jnp.where(i < _NFULL, 4 * i + s, 0)), s))
        for s in range(4)
    ] + [
        pl.BlockSpec(
            (DIM, _BR),
            functools.partial(
                lambda s, i: (0, jnp.where(i < _NFULL, 0, s)), s))
        for s in range(4)
    ],
    out_specs=pl.BlockSpec((_BR, 128), lambda i: (i, 0)),
    out_shape=jax.ShapeDtypeStruct((_PROWS, 128), jnp.float32),
)


def _make_sc_gather():
    mesh = plsc.VectorSubcoreMesh(core_axis_name="c", subcore_axis_name="s")

    @functools.partial(
        pl.kernel,
        mesh=mesh,
        out_type=jax.ShapeDtypeStruct((_OROWS, 128), jnp.float32),
        scratch_types=[
            pltpu.VMEM((_B_PER_W,), jnp.int32),
            pltpu.VMEM((_B_PER_W,), jnp.int32),
            pltpu.VMEM((_B_PER_W, DIM), jnp.float32),
            pltpu.SemaphoreType.DMA,
        ],
        compiler_params=pltpu.CompilerParams(use_tc_tiling_on_sc=False),
    )
    def sc_gather(packed_hbm, idx_hbm, out_hbm, idx_v, row_v, rows_v, sem):
        wid = lax.axis_index("s") * _NC + lax.axis_index("c")
        base = wid * _B_PER_W
        obase = (wid // 4) * _B_PER_W
        ocol = (wid % 4) * 32

        def do_slice(n):
            pltpu.sync_copy(idx_hbm.at[pl.ds(base, n)], idx_v.at[pl.ds(0, n)])

            @pl.loop(0, -(-n // 16))
            def _rows(i):
                off = jnp.minimum(i * 16, n - 16)
                t = idx_v[pl.ds(off, 16)]
                row_v[pl.ds(off, 16)] = (((t >> 14) << 14) + ((t & 4095) << 2)
                                         + ((t >> 12) & 3))

            pltpu.async_copy(packed_hbm.at[row_v.at[pl.ds(0, n)]],
                             rows_v.at[pl.ds(0, n)], sem).wait()
            pltpu.sync_copy(rows_v.at[pl.ds(0, n)],
                            out_hbm.at[pl.ds(obase, n), pl.ds(ocol, DIM)])

        @pl.when(wid < _NW - 1)
        def _full():
            do_slice(_B_PER_W)

        @pl.when(wid == _NW - 1)
        def _last():
            do_slice(_B_LAST)

    return sc_gather


_sc_gather = _make_sc_gather()


def _tc_finish(in_ref, out_ref):
    xt = in_ref[...].T
    out_ref[...] = jnp.concatenate(
        [xt[32 * s:32 * (s + 1), :] for s in range(4)], axis=1)


_finish = pl.pallas_call(
    _tc_finish,
    grid=(_NW // 4,),
    in_specs=[pl.BlockSpec((_B_PER_W, 128), lambda i: (i, 0))],
    out_specs=pl.BlockSpec((DIM, _FBLK), lambda i: (0, i)),
    out_shape=jax.ShapeDtypeStruct((DIM, N_NODES), jnp.float32),
)


def kernel(token_id, table):
    tt = table.T
    tail_tt = jnp.pad(tt[:, _NFULL * _BC:], ((0, 0), (0, (_NFULL + 1) * _BC - VOCAB)))
    packed = _repack(tt, tt, tt, tt, tail_tt, tail_tt, tail_tt, tail_tt)
    packed32 = packed.reshape(_PROWS * 4, DIM)
    outb = _sc_gather(packed32, token_id)
    out_t = _finish(outb)
    return out_t.T
